# Initial kernel scaffold; baseline (speedup 1.0000x reference)
import jax
import jax.numpy as jnp
from jax.experimental import pallas as pl


def kernel(doc_ids, context_ids, sample_ids, paragraph_matrix, word_matrix, outputs):
    raise NotImplementedError("write your pallas kernel here")


# speedup vs baseline: 4.9172x; 4.9172x over previous
import jax
import jax.numpy as jnp
from jax.experimental import pallas as pl
from jax.experimental.pallas import tpu as pltpu


def _dm_kernel(doc_s, ctx_s,
               smp_ref,
               para3, word3,
               out_t,
               o_ref,
               inp_scr):
    TB, D = inp_scr.shape
    C = ctx_s.shape[1]
    S = smp_ref.shape[1]
    NW = out_t.shape[1]
    S_pad = o_ref.shape[1]
    base = pl.program_id(0) * TB

    U = 8

    def chunk(ci, carry):
        rbase = base + ci * U
        rows = []
        for u in range(U):
            r = rbase + u
            acc = para3[doc_s[r]]
            for c in range(C):
                acc = acc + word3[ctx_s[r, c]]
            rows.append(acc)
        blk = jnp.concatenate(rows, axis=0)
        inp_scr[pl.ds(pl.multiple_of(ci * U, U), U), :] = blk
        return carry

    jax.lax.fori_loop(0, TB // U, chunk, 0)

    inputs = inp_scr[...]
    proj = jnp.dot(inputs, out_t[...],
                   preferred_element_type=jnp.float32)

    ids = smp_ref[...]
    iota_n = jax.lax.broadcasted_iota(jnp.int32, (TB, NW), 1)
    lane = jax.lax.broadcasted_iota(jnp.int32, (TB, S_pad), 1)
    res = jnp.zeros((TB, S_pad), jnp.float32)
    for s in range(S):
        oh = (ids[:, s:s + 1] == iota_n).astype(jnp.float32)
        col = jnp.sum(proj * oh, axis=1, keepdims=True)
        res = res + jnp.where(lane == s, col, 0.0)
    o_ref[...] = res


def kernel(doc_ids, context_ids, sample_ids, paragraph_matrix, word_matrix,
           outputs):
    B, C = context_ids.shape
    S = sample_ids.shape[1]
    n_docs, D = paragraph_matrix.shape
    n_words = word_matrix.shape[0]

    TB = 128 if B % 128 == 0 else 8
    B_pad = ((B + TB - 1) // TB) * TB
    S_pad = ((S + 127) // 128) * 128

    pad_b = B_pad - B
    doc = jnp.pad(doc_ids.astype(jnp.int32), ((0, pad_b),))
    ctx = jnp.pad(context_ids.astype(jnp.int32), ((0, pad_b), (0, 0)))
    smp = jnp.pad(sample_ids.astype(jnp.int32), ((0, pad_b), (0, 0)))

    para3 = paragraph_matrix.reshape(n_docs, 1, D)
    word3 = word_matrix.reshape(n_words, 1, D)

    grid_spec = pltpu.PrefetchScalarGridSpec(
        num_scalar_prefetch=2,
        grid=(B_pad // TB,),
        in_specs=[
            pl.BlockSpec((TB, S), lambda i, d, c: (i, 0)),
            pl.BlockSpec((n_docs, 1, D), lambda i, d, c: (0, 0, 0)),
            pl.BlockSpec((n_words, 1, D), lambda i, d, c: (0, 0, 0)),
            pl.BlockSpec((D, n_words), lambda i, d, c: (0, 0)),
        ],
        out_specs=pl.BlockSpec((TB, S_pad), lambda i, d, c: (i, 0)),
        scratch_shapes=[pltpu.VMEM((TB, D), jnp.float32)],
    )

    res = pl.pallas_call(
        _dm_kernel,
        grid_spec=grid_spec,
        out_shape=jax.ShapeDtypeStruct((B_pad, S_pad), jnp.float32),
        compiler_params=pltpu.CompilerParams(
            dimension_semantics=("parallel",),
            vmem_limit_bytes=64 * 1024 * 1024),
    )(doc, ctx, smp, para3, word3, outputs)

    return jnp.squeeze(res[:B, :S])
